# speedup vs baseline: 25.0394x; 25.0394x over previous
import jax
import jax.numpy as jnp
from jax.experimental import pallas as pl
from jax.experimental.pallas import tpu as pltpu

_K = 64
_NEG = -10000000.0
_ROWS = 2048
_COLS = 8192
_BLOCK_R = 256


def _body(x_ref, o_ref):
    x = x_ref[...]
    b = jax.lax.bitcast_convert_type(x, jnp.int32)
    u = jax.lax.bitcast_convert_type(x, jnp.uint32)
    key = jnp.where(b < 0, ~u, u | jnp.uint32(0x80000000))

    res = jnp.zeros((x.shape[0], 1), jnp.uint32)
    for bit in range(31, -1, -1):
        cand = res | jnp.uint32(1 << bit)
        cnt = jnp.sum((key >= cand).astype(jnp.int32), axis=-1, keepdims=True)
        res = jnp.where(cnt >= _K, cand, res)

    was_nonneg = (res & jnp.uint32(0x80000000)) != 0
    ub = jnp.where(was_nonneg, res & jnp.uint32(0x7FFFFFFF), ~res)
    thresh = jax.lax.bitcast_convert_type(ub, jnp.float32)

    xm = jnp.where(x >= thresh, x, jnp.float32(_NEG))
    m = jnp.max(xm, axis=-1, keepdims=True)
    e = jnp.exp(xm - m)
    s = jnp.sum(e, axis=-1, keepdims=True)
    o_ref[...] = e / s


def kernel(X):
    shape = X.shape
    x2 = X.reshape(_ROWS, _COLS)
    out = pl.pallas_call(
        _body,
        grid=(_ROWS // _BLOCK_R,),
        in_specs=[pl.BlockSpec((_BLOCK_R, _COLS), lambda i: (i, 0))],
        out_specs=pl.BlockSpec((_BLOCK_R, _COLS), lambda i: (i, 0)),
        out_shape=jax.ShapeDtypeStruct((_ROWS, _COLS), jnp.float32),
    )(x2)
    return out.reshape(shape)


# speedup vs baseline: 36.6696x; 1.4645x over previous
import jax
import jax.numpy as jnp
from jax.experimental import pallas as pl
from jax.experimental.pallas import tpu as pltpu

_K = 64
_ROWS = 2048
_COLS = 8192
_BLOCK_R = 256


def _pack_i16(x32):
    return pltpu.bitcast(x32, jnp.int16)


def _pack_i32(x16):
    return pltpu.bitcast(x16, jnp.int32)


def _count_pair(cmp):
    c16 = cmp.astype(jnp.int16)
    c32 = _pack_i32(c16)
    return jnp.sum(c32, axis=-1, keepdims=True)


def _lo_bit(bit):
    return jnp.int32(1 << bit)


def _hi_bit(bit):
    return jnp.int32((1 << (bit + 16)) - (1 << 32 if bit == 15 else 0))


_BIAS = -2147450880


def _body(x_ref, o_ref):
    x = x_ref[...]
    r2 = x.shape[0] // 2
    b = jax.lax.bitcast_convert_type(x, jnp.int32)
    u = jax.lax.bitcast_convert_type(x, jnp.uint32)
    key = jnp.where(b < 0, ~u, u | jnp.uint32(0x80000000))
    m = jnp.max(x, axis=-1, keepdims=True)

    hib = ((key >> 16) ^ jnp.uint32(0x8000)).astype(jnp.int16)
    res1 = jnp.zeros((r2, 1), jnp.int32)
    for bit in range(15, -1, -1):
        cand = res1 | (_lo_bit(bit) | _hi_bit(bit))
        cand_b = _pack_i16(cand ^ _BIAS)
        cmp = hib >= cand_b
        s = _count_pair(cmp)
        ge_lo = (s & 0xFFFF) >= _K
        ge_hi = jax.lax.shift_right_logical(s, 16) >= _K
        res1 = (res1
                | jnp.where(ge_lo, _lo_bit(bit), 0)
                | jnp.where(ge_hi, _hi_bit(bit), 0))
    res1_b = _pack_i16(res1 ^ _BIAS)
    s = _count_pair(hib > res1_b)
    k2_lo = _K - (s & 0xFFFF)
    k2_hi = _K - jax.lax.shift_right_logical(s, 16)

    boundary = hib == res1_b
    lob = jnp.where(
        boundary,
        ((key ^ jnp.uint32(0x8000)) & jnp.uint32(0xFFFF)).astype(jnp.int16),
        jnp.int16(-32768))
    res2 = jnp.zeros((r2, 1), jnp.int32)
    for bit in range(15, -1, -1):
        cand = res2 | (_lo_bit(bit) | _hi_bit(bit))
        cand_b = _pack_i16(cand ^ _BIAS)
        cmp = lob >= cand_b
        s = _count_pair(cmp)
        ge_lo = (s & 0xFFFF) >= k2_lo
        ge_hi = jax.lax.shift_right_logical(s, 16) >= k2_hi
        res2 = (res2
                | jnp.where(ge_lo, _lo_bit(bit), 0)
                | jnp.where(ge_hi, _hi_bit(bit), 0))

    hi16 = _pack_i16(res1).astype(jnp.int32) & 0xFFFF
    lo16 = _pack_i16(res2).astype(jnp.int32) & 0xFFFF
    T = jax.lax.bitcast_convert_type((hi16 << 16) | lo16, jnp.uint32)
    was_nonneg = (T & jnp.uint32(0x80000000)) != 0
    ub = jnp.where(was_nonneg, T & jnp.uint32(0x7FFFFFFF), ~T)
    thresh = jax.lax.bitcast_convert_type(ub, jnp.float32)

    e = jnp.where(x >= thresh, jnp.exp(x - m), jnp.float32(0.0))
    s = jnp.sum(e, axis=-1, keepdims=True)
    o_ref[...] = e / s


def kernel(X):
    shape = X.shape
    x2 = X.reshape(_ROWS, _COLS)
    out = pl.pallas_call(
        _body,
        grid=(_ROWS // _BLOCK_R,),
        in_specs=[pl.BlockSpec((_BLOCK_R, _COLS), lambda i: (i, 0))],
        out_specs=pl.BlockSpec((_BLOCK_R, _COLS), lambda i: (i, 0)),
        out_shape=jax.ShapeDtypeStruct((_ROWS, _COLS), jnp.float32),
    )(x2)
    return out.reshape(shape)
